# baseline (speedup 1.0000x reference)
import functools
import jax
import jax.numpy as jnp
from jax import lax
from jax.experimental import pallas as pl
from jax.experimental.pallas import tpu as pltpu
from jax.experimental.pallas import tpu_sc as plsc

CACHELINE = 64
THRESHOLD = 0.05
_NC = 2
_NS = 16
_NW = _NC * _NS
_L = 16
GROUP_ELEMS = CACHELINE * _L
PAIR_ELEMS = 2 * GROUP_ELEMS
PAIR_W = 2 * _L
BLK = 8


def _perm(v, idx):
    return lax.gather(
        v, idx[:, None],
        dimension_numbers=lax.GatherDimensionNumbers(
            offset_dims=(), collapsed_slice_dims=(0,), start_index_map=(0,)),
        slice_sizes=(1,),
        mode=lax.GatherScatterMode.PROMISE_IN_BOUNDS,
        unique_indices=True, indices_are_sorted=False)


def _xpose16(v, lane):
    for d in (1, 2, 4, 8):
        idx = lane ^ d
        keep = (lane & d) == 0
        nv = list(v)
        for i in range(16):
            if i & d:
                continue
            p = i | d
            a, b = v[i], v[p]
            nv[i] = jnp.where(keep, a, _perm(b, idx))
            nv[p] = jnp.where(keep, _perm(a, idx), b)
        v = nv
    return v


def _make_cluster_call(num_groups: int, rot: int = 0):
    num_pairs = num_groups // 2
    base_ppw = num_pairs // _NW
    extra = num_pairs % _NW
    tmax = base_ppw // 2
    max_left = base_ppw % 2 + (1 if extra else 0)
    mesh = plsc.VectorSubcoreMesh(core_axis_name="c", subcore_axis_name="s")

    @functools.partial(
        pl.kernel,
        out_type=jax.ShapeDtypeStruct((num_groups * GROUP_ELEMS,), jnp.float32),
        mesh=mesh,
        scratch_types=[
            pltpu.VMEM((PAIR_ELEMS,), jnp.float32),
            pltpu.VMEM((PAIR_ELEMS,), jnp.float32),
            pltpu.VMEM((PAIR_ELEMS,), jnp.float32),
            pltpu.VMEM((GROUP_ELEMS,), jnp.float32),
            pltpu.VMEM((GROUP_ELEMS,), jnp.float32),
            pltpu.SemaphoreType.DMA,
            pltpu.SemaphoreType.DMA,
            pltpu.SemaphoreType.DMA,
            pltpu.SemaphoreType.DMA,
        ],
    )
    def cluster(x_hbm, out_hbm, sb0, sb1, wb, ba, bc, si0, si1, so0, so1):
        wid = lax.axis_index("s") * _NC + lax.axis_index("c")
        eid = jax.lax.rem(wid - rot + _NW, _NW)
        ppw = base_ppw + jnp.where(eid < extra, 1, 0)
        pbase = eid * base_ppw + jnp.minimum(eid, extra)
        lane = lax.iota(jnp.int32, _L)
        inf_row = jnp.full((_L,), jnp.inf, jnp.float32) * jnp.where(
            wid >= 0, jnp.float32(1.0), jnp.float32(0.0)
        )

        def hbm_pair(q):
            return x_hbm.at[pl.ds((pbase + q) * PAIR_ELEMS, PAIR_ELEMS)]

        def out_pair(q):
            return out_hbm.at[pl.ds((pbase + q) * PAIR_ELEMS, PAIR_ELEMS)]

        def wait_pair(ref, sem):
            pltpu.make_async_copy(x_hbm.at[pl.ds(0, PAIR_ELEMS)], ref, sem).wait()

        def compute_pair(sb):
            def tile_in(t, carry):
                lbase = (t // 4) * _L * CACHELINE + (t % 4) * _L
                rbase = (t % 4) * _L * PAIR_W + (t // 4) * _L
                v = [sb[pl.ds(lbase + l * CACHELINE, _L)] for l in range(_L)]
                v = _xpose16(v, lane)
                for js in range(_L):
                    wb[pl.ds(rbase + js * PAIR_W, _L)] = v[js]
                return carry

            lax.fori_loop(0, 8, tile_in, 0)

            offb = _L

            def block_body(B, carry):
                j0 = B * BLK
                xjs_a = [wb[pl.ds((j0 + u) * PAIR_W, _L)] for u in range(BLK)]
                xjs_b = [wb[pl.ds((j0 + u) * PAIR_W + offb, _L)]
                         for u in range(BLK)]
                start = CACHELINE - j0

                def make_sweep(bref, xjs):
                    def sweep(t, kc):
                        r = list(kc)
                        rowb = (start + t * 4) * _L
                        for u in range(4):
                            bv = bref[pl.ds(rowb + u * _L, _L)]
                            for q in range(BLK):
                                r[q] = jnp.where(
                                    jnp.abs(bv - xjs[q]) < THRESHOLD,
                                    bv, r[q],
                                )
                        return tuple(r)
                    return sweep

                res_a = list(lax.fori_loop(
                    0, 2 * B, make_sweep(ba, xjs_a), tuple([inf_row] * BLK)))
                res_b = list(lax.fori_loop(
                    0, 2 * B, make_sweep(bc, xjs_b), tuple([inf_row] * BLK)))

                banda = []
                bandb = []
                for u in range(BLK):
                    j = j0 + u
                    xja = xjs_a[u]
                    xjb = xjs_b[u]
                    owna = xja
                    ownb = xjb
                    for w in reversed(range(u)):
                        owna = jnp.where(
                            jnp.abs(banda[w] - xja) < THRESHOLD, banda[w], owna
                        )
                        ownb = jnp.where(
                            jnp.abs(bandb[w] - xjb) < THRESHOLD, bandb[w], ownb
                        )
                    ra = jnp.where(res_a[u] == jnp.inf, owna, res_a[u])
                    rb = jnp.where(res_b[u] == jnp.inf, ownb, res_b[u])
                    nba = jnp.where(ra != xja, jnp.inf, xja)
                    nbb = jnp.where(rb != xjb, jnp.inf, xjb)
                    banda.append(nba)
                    bandb.append(nbb)
                    ba[pl.ds((CACHELINE - 1 - j) * _L, _L)] = nba
                    bc[pl.ds((CACHELINE - 1 - j) * _L, _L)] = nbb
                    wb[pl.ds(j * PAIR_W, _L)] = ra
                    wb[pl.ds(j * PAIR_W + offb, _L)] = rb
                return carry

            lax.fori_loop(0, CACHELINE // BLK, block_body, 0)

            def tile_out(t, carry):
                lbase = (t // 4) * _L * CACHELINE + (t % 4) * _L
                rbase = (t % 4) * _L * PAIR_W + (t // 4) * _L
                v = [wb[pl.ds(rbase + js * PAIR_W, _L)] for js in range(_L)]
                v = _xpose16(v, lane)
                for l in range(_L):
                    sb[pl.ds(lbase + l * CACHELINE, _L)] = v[l]
                return carry

            lax.fori_loop(0, 8, tile_out, 0)

        if tmax > 0:
            pltpu.async_copy(hbm_pair(0), sb0, si0)

            def step(t, carry):
                for b in range(2):
                    q = 2 * t + b
                    buf, sem_i, sem_o = (sb0, si0, so0) if b == 0 else (sb1, si1, so1)
                    obuf, osem_i, osem_o = (sb1, si1, so1) if b == 0 else (sb0, si0, so0)

                    if b == 0:
                        @pl.when(t > 0)
                        def _():
                            wait_pair(obuf, osem_o)
                    else:
                        wait_pair(obuf, osem_o)
                    nq = jnp.minimum(q + 1, ppw - 1)
                    pltpu.async_copy(hbm_pair(nq), obuf, osem_i)
                    wait_pair(buf, sem_i)
                    compute_pair(buf)
                    pltpu.async_copy(buf, out_pair(q), sem_o)
                return carry

            lax.fori_loop(0, tmax, step, 0)
            wait_pair(sb1, so1)
            wait_pair(sb0, si0)

        for rr in range(max_left):
            @pl.when(ppw > 2 * tmax + rr)
            def _():
                q = 2 * tmax + rr
                pltpu.sync_copy(hbm_pair(q), sb0)
                compute_pair(sb0)
                pltpu.sync_copy(sb0, out_pair(q))

    return cluster


def kernel(x):
    shape = x.shape
    flat = x.reshape(-1)
    n = flat.shape[0]
    m = n // CACHELINE

    num_groups = m // _L
    paired_groups = (num_groups // 2) * 2
    covered = paired_groups * GROUP_ELEMS

    if paired_groups >= 16 * _NW:
        small = ((paired_groups // 9) // 2) * 2
        mid = ((paired_groups - 2 * small) // 4) * 2
        chunks = [small, mid, mid, paired_groups - 2 * small - 2 * mid + small]
    else:
        chunks = [paired_groups]
    outs = []
    off = 0
    rot = 0
    for cg in chunks:
        outs.append(
            _make_cluster_call(cg, rot)(
                lax.slice(flat, (off,), (off + cg * GROUP_ELEMS,))
            )
        )
        rot += ((cg // 2) % _NW) or 0
        off += cg * GROUP_ELEMS
    out = outs[0] if len(outs) == 1 else jnp.concatenate(outs)

    if covered != m * CACHELINE:
        tail = flat[covered: m * CACHELINE]
        tpad = PAIR_ELEMS - tail.shape[0]
        tarr = jnp.concatenate([tail, jnp.zeros((tpad,), jnp.float32)])
        tout = _make_cluster_call(2)(tarr)
        out = jnp.concatenate([out, tout[: tail.shape[0]]])
    if m * CACHELINE != n:
        out = jnp.concatenate([out, flat[m * CACHELINE:]])
    return out.reshape(shape)


# speedup vs baseline: 1.1927x; 1.1927x over previous
import functools
import jax
import jax.numpy as jnp
from jax import lax
from jax.experimental import pallas as pl
from jax.experimental.pallas import tpu as pltpu
from jax.experimental.pallas import tpu_sc as plsc

CACHELINE = 64
THRESHOLD = 0.05
_NC = 2
_NS = 16
_NW = _NC * _NS
_L = 16
GROUP_ELEMS = CACHELINE * _L
PAIR_ELEMS = 2 * GROUP_ELEMS
PAIR_W = 2 * _L
BLK = 8


def _perm(v, idx):
    return lax.gather(
        v, idx[:, None],
        dimension_numbers=lax.GatherDimensionNumbers(
            offset_dims=(), collapsed_slice_dims=(0,), start_index_map=(0,)),
        slice_sizes=(1,),
        mode=lax.GatherScatterMode.PROMISE_IN_BOUNDS,
        unique_indices=True, indices_are_sorted=False)


def _xpose16(v, lane):
    for d in (1, 2, 4, 8):
        idx = lane ^ d
        keep = (lane & d) == 0
        nv = list(v)
        for i in range(16):
            if i & d:
                continue
            p = i | d
            a, b = v[i], v[p]
            nv[i] = jnp.where(keep, a, _perm(b, idx))
            nv[p] = jnp.where(keep, _perm(a, idx), b)
        v = nv
    return v


def _make_cluster_call(num_groups: int):
    num_pairs = num_groups // 2
    base_ppw = num_pairs // _NW
    extra = num_pairs % _NW
    tmax = base_ppw // 2
    max_left = base_ppw % 2 + (1 if extra else 0)
    mesh = plsc.VectorSubcoreMesh(core_axis_name="c", subcore_axis_name="s")

    @functools.partial(
        pl.kernel,
        out_type=jax.ShapeDtypeStruct((num_groups * GROUP_ELEMS,), jnp.float32),
        mesh=mesh,
        scratch_types=[
            pltpu.VMEM((PAIR_ELEMS,), jnp.float32),
            pltpu.VMEM((PAIR_ELEMS,), jnp.float32),
            pltpu.VMEM((PAIR_ELEMS,), jnp.float32),
            pltpu.VMEM((GROUP_ELEMS,), jnp.float32),
            pltpu.VMEM((GROUP_ELEMS,), jnp.float32),
            pltpu.SemaphoreType.DMA,
            pltpu.SemaphoreType.DMA,
            pltpu.SemaphoreType.DMA,
            pltpu.SemaphoreType.DMA,
        ],
    )
    def cluster(x_hbm, out_hbm, sb0, sb1, wb, ba, bc, si0, si1, so0, so1):
        wid = lax.axis_index("s") * _NC + lax.axis_index("c")
        ppw = base_ppw + jnp.where(wid < extra, 1, 0)
        pbase = wid * base_ppw + jnp.minimum(wid, extra)
        lane = lax.iota(jnp.int32, _L)
        inf_row = jnp.full((_L,), jnp.inf, jnp.float32) * jnp.where(
            wid >= 0, jnp.float32(1.0), jnp.float32(0.0)
        )

        def hbm_pair(q):
            return x_hbm.at[pl.ds((pbase + q) * PAIR_ELEMS, PAIR_ELEMS)]

        def out_pair(q):
            return out_hbm.at[pl.ds((pbase + q) * PAIR_ELEMS, PAIR_ELEMS)]

        def wait_pair(ref, sem):
            pltpu.make_async_copy(x_hbm.at[pl.ds(0, PAIR_ELEMS)], ref, sem).wait()

        def compute_pair(sb):
            def tile_in(t, carry):
                lbase = (t // 4) * _L * CACHELINE + (t % 4) * _L
                rbase = (t % 4) * _L * PAIR_W + (t // 4) * _L
                v = [sb[pl.ds(lbase + l * CACHELINE, _L)] for l in range(_L)]
                v = _xpose16(v, lane)
                for js in range(_L):
                    wb[pl.ds(rbase + js * PAIR_W, _L)] = v[js]
                return carry

            lax.fori_loop(0, 8, tile_in, 0)

            offb = _L

            def block_body(B, carry):
                j0 = B * BLK
                xjs_a = [wb[pl.ds((j0 + u) * PAIR_W, _L)] for u in range(BLK)]
                xjs_b = [wb[pl.ds((j0 + u) * PAIR_W + offb, _L)]
                         for u in range(BLK)]
                start = CACHELINE - j0

                def make_sweep(bref, xjs):
                    def sweep(t, kc):
                        r = list(kc)
                        rowb = (start + t * 4) * _L
                        for u in range(4):
                            bv = bref[pl.ds(rowb + u * _L, _L)]
                            for q in range(BLK):
                                r[q] = jnp.where(
                                    jnp.abs(bv - xjs[q]) < THRESHOLD,
                                    bv, r[q],
                                )
                        return tuple(r)
                    return sweep

                res_a = list(lax.fori_loop(
                    0, 2 * B, make_sweep(ba, xjs_a), tuple([inf_row] * BLK)))
                res_b = list(lax.fori_loop(
                    0, 2 * B, make_sweep(bc, xjs_b), tuple([inf_row] * BLK)))

                banda = []
                bandb = []
                for u in range(BLK):
                    j = j0 + u
                    xja = xjs_a[u]
                    xjb = xjs_b[u]
                    owna = xja
                    ownb = xjb
                    for w in reversed(range(u)):
                        owna = jnp.where(
                            jnp.abs(banda[w] - xja) < THRESHOLD, banda[w], owna
                        )
                        ownb = jnp.where(
                            jnp.abs(bandb[w] - xjb) < THRESHOLD, bandb[w], ownb
                        )
                    ra = jnp.where(res_a[u] == jnp.inf, owna, res_a[u])
                    rb = jnp.where(res_b[u] == jnp.inf, ownb, res_b[u])
                    nba = jnp.where(ra != xja, jnp.inf, xja)
                    nbb = jnp.where(rb != xjb, jnp.inf, xjb)
                    banda.append(nba)
                    bandb.append(nbb)
                    ba[pl.ds((CACHELINE - 1 - j) * _L, _L)] = nba
                    bc[pl.ds((CACHELINE - 1 - j) * _L, _L)] = nbb
                    wb[pl.ds(j * PAIR_W, _L)] = ra
                    wb[pl.ds(j * PAIR_W + offb, _L)] = rb
                return carry

            lax.fori_loop(0, CACHELINE // BLK, block_body, 0)

            def tile_out(t, carry):
                lbase = (t // 4) * _L * CACHELINE + (t % 4) * _L
                rbase = (t % 4) * _L * PAIR_W + (t // 4) * _L
                v = [wb[pl.ds(rbase + js * PAIR_W, _L)] for js in range(_L)]
                v = _xpose16(v, lane)
                for l in range(_L):
                    sb[pl.ds(lbase + l * CACHELINE, _L)] = v[l]
                return carry

            lax.fori_loop(0, 8, tile_out, 0)

        if tmax > 0:
            pltpu.async_copy(hbm_pair(0), sb0, si0)

            def step(t, carry):
                for b in range(2):
                    q = 2 * t + b
                    buf, sem_i, sem_o = (sb0, si0, so0) if b == 0 else (sb1, si1, so1)
                    obuf, osem_i, osem_o = (sb1, si1, so1) if b == 0 else (sb0, si0, so0)

                    if b == 0:
                        @pl.when(t > 0)
                        def _():
                            wait_pair(obuf, osem_o)
                    else:
                        wait_pair(obuf, osem_o)
                    nq = jnp.minimum(q + 1, ppw - 1)
                    pltpu.async_copy(hbm_pair(nq), obuf, osem_i)
                    wait_pair(buf, sem_i)
                    compute_pair(buf)
                    pltpu.async_copy(buf, out_pair(q), sem_o)
                return carry

            lax.fori_loop(0, tmax, step, 0)
            wait_pair(sb1, so1)
            wait_pair(sb0, si0)

        for rr in range(max_left):
            @pl.when(ppw > 2 * tmax + rr)
            def _():
                q = 2 * tmax + rr
                pltpu.sync_copy(hbm_pair(q), sb0)
                compute_pair(sb0)
                pltpu.sync_copy(sb0, out_pair(q))

    return cluster


def kernel(x):
    shape = x.shape
    flat = x.reshape(-1)
    n = flat.shape[0]
    m = n // CACHELINE

    num_groups = m // _L
    paired_groups = (num_groups // 2) * 2
    covered = paired_groups * GROUP_ELEMS

    if paired_groups >= 4 * _NW and paired_groups % 4 == 0:
        half = paired_groups // 2
        h_elems = half * GROUP_ELEMS
        o1 = _make_cluster_call(half)(lax.slice(flat, (0,), (h_elems,)))
        o2 = _make_cluster_call(half)(lax.slice(flat, (h_elems,), (2 * h_elems,)))
        out = jnp.concatenate([o1, o2])
    else:
        out = _make_cluster_call(paired_groups)(flat[:covered])

    if covered != m * CACHELINE:
        tail = flat[covered: m * CACHELINE]
        tpad = PAIR_ELEMS - tail.shape[0]
        tarr = jnp.concatenate([tail, jnp.zeros((tpad,), jnp.float32)])
        tout = _make_cluster_call(2)(tarr)
        out = jnp.concatenate([out, tout[: tail.shape[0]]])
    if m * CACHELINE != n:
        out = jnp.concatenate([out, flat[m * CACHELINE:]])
    return out.reshape(shape)


# speedup vs baseline: 1.1987x; 1.0050x over previous
import functools
import jax
import jax.numpy as jnp
from jax import lax
from jax.experimental import pallas as pl
from jax.experimental.pallas import tpu as pltpu
from jax.experimental.pallas import tpu_sc as plsc

CACHELINE = 64
THRESHOLD = 0.05
_NC = 2
_NS = 16
_NW = _NC * _NS
_L = 16
GROUP_ELEMS = CACHELINE * _L
PAIR_ELEMS = 2 * GROUP_ELEMS
PAIR_W = 2 * _L
BLK = 8


def _perm(v, idx):
    return lax.gather(
        v, idx[:, None],
        dimension_numbers=lax.GatherDimensionNumbers(
            offset_dims=(), collapsed_slice_dims=(0,), start_index_map=(0,)),
        slice_sizes=(1,),
        mode=lax.GatherScatterMode.PROMISE_IN_BOUNDS,
        unique_indices=True, indices_are_sorted=False)


def _xpose16(v, lane):
    for d in (1, 2, 4, 8):
        idx = lane ^ d
        keep = (lane & d) == 0
        nv = list(v)
        for i in range(16):
            if i & d:
                continue
            p = i | d
            a, b = v[i], v[p]
            nv[i] = jnp.where(keep, a, _perm(b, idx))
            nv[p] = jnp.where(keep, _perm(a, idx), b)
        v = nv
    return v


def _make_cluster_call(num_groups: int):
    num_pairs = num_groups // 2
    base_ppw = num_pairs // _NW
    extra = num_pairs % _NW
    tmax = base_ppw // 2
    max_left = base_ppw % 2 + (1 if extra else 0)
    mesh = plsc.VectorSubcoreMesh(core_axis_name="c", subcore_axis_name="s")

    @functools.partial(
        pl.kernel,
        out_type=jax.ShapeDtypeStruct((num_groups * GROUP_ELEMS,), jnp.float32),
        mesh=mesh,
        scratch_types=[
            pltpu.VMEM((PAIR_ELEMS,), jnp.float32),
            pltpu.VMEM((PAIR_ELEMS,), jnp.float32),
            pltpu.VMEM((PAIR_ELEMS,), jnp.float32),
            pltpu.VMEM((GROUP_ELEMS,), jnp.float32),
            pltpu.VMEM((GROUP_ELEMS,), jnp.float32),
            pltpu.SemaphoreType.DMA,
            pltpu.SemaphoreType.DMA,
            pltpu.SemaphoreType.DMA,
            pltpu.SemaphoreType.DMA,
        ],
    )
    def cluster(x_hbm, out_hbm, sb0, sb1, wb, ba, bc, si0, si1, so0, so1):
        wid = lax.axis_index("s") * _NC + lax.axis_index("c")
        ppw = base_ppw + jnp.where(wid < extra, 1, 0)
        pbase = wid * base_ppw + jnp.minimum(wid, extra)
        lane = lax.iota(jnp.int32, _L)
        inf_row = jnp.full((_L,), jnp.inf, jnp.float32) * jnp.where(
            wid >= 0, jnp.float32(1.0), jnp.float32(0.0)
        )

        def hbm_pair(q):
            return x_hbm.at[pl.ds((pbase + q) * PAIR_ELEMS, PAIR_ELEMS)]

        def out_pair(q):
            return out_hbm.at[pl.ds((pbase + q) * PAIR_ELEMS, PAIR_ELEMS)]

        def wait_pair(ref, sem):
            pltpu.make_async_copy(x_hbm.at[pl.ds(0, PAIR_ELEMS)], ref, sem).wait()

        def compute_pair(sb):
            def tile_in(t, carry):
                lbase = (t // 4) * _L * CACHELINE + (t % 4) * _L
                rbase = (t % 4) * _L * PAIR_W + (t // 4) * _L
                v = [sb[pl.ds(lbase + l * CACHELINE, _L)] for l in range(_L)]
                v = _xpose16(v, lane)
                for js in range(_L):
                    wb[pl.ds(rbase + js * PAIR_W, _L)] = v[js]
                return carry

            lax.fori_loop(0, 8, tile_in, 0)

            offb = _L

            def block_body(B, carry):
                j0 = B * BLK
                xjs_a = [wb[pl.ds((j0 + u) * PAIR_W, _L)] for u in range(BLK)]
                xjs_b = [wb[pl.ds((j0 + u) * PAIR_W + offb, _L)]
                         for u in range(BLK)]
                start = CACHELINE - j0

                def make_sweep(bref, xjs):
                    def sweep(t, kc):
                        r = list(kc)
                        rowb = (start + t * 4) * _L
                        for u in range(4):
                            bv = bref[pl.ds(rowb + u * _L, _L)]
                            for q in range(BLK):
                                r[q] = jnp.where(
                                    jnp.abs(bv - xjs[q]) < THRESHOLD,
                                    bv, r[q],
                                )
                        return tuple(r)
                    return sweep

                res_a = list(lax.fori_loop(
                    0, 2 * B, make_sweep(ba, xjs_a), tuple([inf_row] * BLK)))
                res_b = list(lax.fori_loop(
                    0, 2 * B, make_sweep(bc, xjs_b), tuple([inf_row] * BLK)))

                banda = []
                bandb = []
                for u in range(BLK):
                    j = j0 + u
                    xja = xjs_a[u]
                    xjb = xjs_b[u]
                    owna = xja
                    ownb = xjb
                    for w in reversed(range(u)):
                        owna = jnp.where(
                            jnp.abs(banda[w] - xja) < THRESHOLD, banda[w], owna
                        )
                        ownb = jnp.where(
                            jnp.abs(bandb[w] - xjb) < THRESHOLD, bandb[w], ownb
                        )
                    ra = jnp.where(res_a[u] == jnp.inf, owna, res_a[u])
                    rb = jnp.where(res_b[u] == jnp.inf, ownb, res_b[u])
                    nba = jnp.where(ra != xja, jnp.inf, xja)
                    nbb = jnp.where(rb != xjb, jnp.inf, xjb)
                    banda.append(nba)
                    bandb.append(nbb)
                    ba[pl.ds((CACHELINE - 1 - j) * _L, _L)] = nba
                    bc[pl.ds((CACHELINE - 1 - j) * _L, _L)] = nbb
                    wb[pl.ds(j * PAIR_W, _L)] = ra
                    wb[pl.ds(j * PAIR_W + offb, _L)] = rb
                return carry

            lax.fori_loop(0, CACHELINE // BLK, block_body, 0)

            def tile_out(t, carry):
                lbase = (t // 4) * _L * CACHELINE + (t % 4) * _L
                rbase = (t % 4) * _L * PAIR_W + (t // 4) * _L
                v = [wb[pl.ds(rbase + js * PAIR_W, _L)] for js in range(_L)]
                v = _xpose16(v, lane)
                for l in range(_L):
                    sb[pl.ds(lbase + l * CACHELINE, _L)] = v[l]
                return carry

            lax.fori_loop(0, 8, tile_out, 0)

        if tmax > 0:
            pltpu.async_copy(hbm_pair(0), sb0, si0)

            def step(t, carry):
                for b in range(2):
                    q = 2 * t + b
                    buf, sem_i, sem_o = (sb0, si0, so0) if b == 0 else (sb1, si1, so1)
                    obuf, osem_i, osem_o = (sb1, si1, so1) if b == 0 else (sb0, si0, so0)

                    if b == 0:
                        @pl.when(t > 0)
                        def _():
                            wait_pair(obuf, osem_o)
                    else:
                        wait_pair(obuf, osem_o)
                    nq = jnp.minimum(q + 1, ppw - 1)
                    pltpu.async_copy(hbm_pair(nq), obuf, osem_i)
                    wait_pair(buf, sem_i)
                    compute_pair(buf)
                    pltpu.async_copy(buf, out_pair(q), sem_o)
                return carry

            lax.fori_loop(0, tmax, step, 0)
            wait_pair(sb1, so1)
            wait_pair(sb0, si0)

        for rr in range(max_left):
            @pl.when(ppw > 2 * tmax + rr)
            def _():
                q = 2 * tmax + rr
                pltpu.sync_copy(hbm_pair(q), sb0)
                compute_pair(sb0)
                pltpu.sync_copy(sb0, out_pair(q))

    return cluster


def kernel(x):
    shape = x.shape
    flat = x.reshape(-1)
    n = flat.shape[0]
    m = n // CACHELINE

    num_groups = m // _L
    paired_groups = (num_groups // 2) * 2
    covered = paired_groups * GROUP_ELEMS

    out = _make_cluster_call(paired_groups)(flat[:covered])

    if covered != m * CACHELINE:
        tail = flat[covered: m * CACHELINE]
        tpad = PAIR_ELEMS - tail.shape[0]
        tarr = jnp.concatenate([tail, jnp.zeros((tpad,), jnp.float32)])
        tout = _make_cluster_call(2)(tarr)
        out = jnp.concatenate([out, tout[: tail.shape[0]]])
    if m * CACHELINE != n:
        out = jnp.concatenate([out, flat[m * CACHELINE:]])
    return out.reshape(shape)


# speedup vs baseline: 1.2193x; 1.0172x over previous
import functools
import jax
import jax.numpy as jnp
from jax import lax
from jax.experimental import pallas as pl
from jax.experimental.pallas import tpu as pltpu
from jax.experimental.pallas import tpu_sc as plsc

CACHELINE = 64
THRESHOLD = 0.05
_NC = 2
_NS = 16
_NW = _NC * _NS
_L = 16
GROUP_ELEMS = CACHELINE * _L
PAIR_ELEMS = 2 * GROUP_ELEMS
PAIR_W = 2 * _L
BLK = 8


def _perm(v, idx):
    return lax.gather(
        v, idx[:, None],
        dimension_numbers=lax.GatherDimensionNumbers(
            offset_dims=(), collapsed_slice_dims=(0,), start_index_map=(0,)),
        slice_sizes=(1,),
        mode=lax.GatherScatterMode.PROMISE_IN_BOUNDS,
        unique_indices=True, indices_are_sorted=False)


def _xpose16(v, lane):
    for d in (1, 2, 4, 8):
        idx = lane ^ d
        keep = (lane & d) == 0
        nv = list(v)
        for i in range(16):
            if i & d:
                continue
            p = i | d
            a, b = v[i], v[p]
            nv[i] = jnp.where(keep, a, _perm(b, idx))
            nv[p] = jnp.where(keep, _perm(a, idx), b)
        v = nv
    return v


def _make_cluster_call(num_groups: int):
    num_pairs = num_groups // 2
    base_ppw = num_pairs // _NW
    extra = num_pairs % _NW
    tmax = base_ppw // 2
    max_left = base_ppw % 2 + (1 if extra else 0)
    mesh = plsc.VectorSubcoreMesh(core_axis_name="c", subcore_axis_name="s")

    @functools.partial(
        pl.kernel,
        out_type=jax.ShapeDtypeStruct((num_groups * GROUP_ELEMS,), jnp.float32),
        mesh=mesh,
        scratch_types=[
            pltpu.VMEM((PAIR_ELEMS,), jnp.float32),
            pltpu.VMEM((PAIR_ELEMS,), jnp.float32),
            pltpu.VMEM((PAIR_ELEMS,), jnp.float32),
            pltpu.VMEM((PAIR_ELEMS,), jnp.float32),
            pltpu.VMEM((PAIR_ELEMS,), jnp.float32),
            pltpu.VMEM((GROUP_ELEMS,), jnp.float32),
            pltpu.VMEM((GROUP_ELEMS,), jnp.float32),
            pltpu.SemaphoreType.DMA,
            pltpu.SemaphoreType.DMA,
            pltpu.SemaphoreType.DMA,
            pltpu.SemaphoreType.DMA,
        ],
    )
    def cluster(x_hbm, out_hbm, sb0, sb1, ob0, ob1, wb, ba, bc, si0, si1, so0, so1):
        wid = lax.axis_index("s") * _NC + lax.axis_index("c")
        ppw = base_ppw + jnp.where(wid < extra, 1, 0)
        pbase = wid * base_ppw + jnp.minimum(wid, extra)
        lane = lax.iota(jnp.int32, _L)
        inf_row = jnp.full((_L,), jnp.inf, jnp.float32) * jnp.where(
            wid >= 0, jnp.float32(1.0), jnp.float32(0.0)
        )

        def hbm_pair(q):
            return x_hbm.at[pl.ds((pbase + q) * PAIR_ELEMS, PAIR_ELEMS)]

        def out_pair(q):
            return out_hbm.at[pl.ds((pbase + q) * PAIR_ELEMS, PAIR_ELEMS)]

        def wait_pair(ref, sem):
            pltpu.make_async_copy(x_hbm.at[pl.ds(0, PAIR_ELEMS)], ref, sem).wait()

        def compute_pair(sb):
            def tile_in(t, carry):
                lbase = (t // 4) * _L * CACHELINE + (t % 4) * _L
                rbase = (t % 4) * _L * PAIR_W + (t // 4) * _L
                v = [sb[pl.ds(lbase + l * CACHELINE, _L)] for l in range(_L)]
                v = _xpose16(v, lane)
                for js in range(_L):
                    wb[pl.ds(rbase + js * PAIR_W, _L)] = v[js]
                return carry

            lax.fori_loop(0, 8, tile_in, 0)

            offb = _L

            def block_body(B, carry):
                j0 = B * BLK
                xjs_a = [wb[pl.ds((j0 + u) * PAIR_W, _L)] for u in range(BLK)]
                xjs_b = [wb[pl.ds((j0 + u) * PAIR_W + offb, _L)]
                         for u in range(BLK)]
                start = CACHELINE - j0

                def make_sweep(bref, xjs):
                    def sweep(t, kc):
                        r = list(kc)
                        rowb = (start + t * 4) * _L
                        for u in range(4):
                            bv = bref[pl.ds(rowb + u * _L, _L)]
                            for q in range(BLK):
                                r[q] = jnp.where(
                                    jnp.abs(bv - xjs[q]) < THRESHOLD,
                                    bv, r[q],
                                )
                        return tuple(r)
                    return sweep

                res_a = list(lax.fori_loop(
                    0, 2 * B, make_sweep(ba, xjs_a), tuple([inf_row] * BLK)))
                res_b = list(lax.fori_loop(
                    0, 2 * B, make_sweep(bc, xjs_b), tuple([inf_row] * BLK)))

                banda = []
                bandb = []
                for u in range(BLK):
                    j = j0 + u
                    xja = xjs_a[u]
                    xjb = xjs_b[u]
                    owna = xja
                    ownb = xjb
                    for w in reversed(range(u)):
                        owna = jnp.where(
                            jnp.abs(banda[w] - xja) < THRESHOLD, banda[w], owna
                        )
                        ownb = jnp.where(
                            jnp.abs(bandb[w] - xjb) < THRESHOLD, bandb[w], ownb
                        )
                    ra = jnp.where(res_a[u] == jnp.inf, owna, res_a[u])
                    rb = jnp.where(res_b[u] == jnp.inf, ownb, res_b[u])
                    nba = jnp.where(ra != xja, jnp.inf, xja)
                    nbb = jnp.where(rb != xjb, jnp.inf, xjb)
                    banda.append(nba)
                    bandb.append(nbb)
                    ba[pl.ds((CACHELINE - 1 - j) * _L, _L)] = nba
                    bc[pl.ds((CACHELINE - 1 - j) * _L, _L)] = nbb
                    wb[pl.ds(j * PAIR_W, _L)] = ra
                    wb[pl.ds(j * PAIR_W + offb, _L)] = rb
                return carry

            lax.fori_loop(0, CACHELINE // BLK, block_body, 0)

        def emit_pair(ob):
            def tile_out(t, carry):
                lbase = (t // 4) * _L * CACHELINE + (t % 4) * _L
                rbase = (t % 4) * _L * PAIR_W + (t // 4) * _L
                v = [wb[pl.ds(rbase + js * PAIR_W, _L)] for js in range(_L)]
                v = _xpose16(v, lane)
                for l in range(_L):
                    ob[pl.ds(lbase + l * CACHELINE, _L)] = v[l]
                return carry

            lax.fori_loop(0, 8, tile_out, 0)

        if tmax > 0:
            pltpu.async_copy(hbm_pair(0), sb0, si0)

            def step(t, carry):
                for b in range(2):
                    q = 2 * t + b
                    buf, sem_i = (sb0, si0) if b == 0 else (sb1, si1)
                    obuf, sem_o = (ob0, so0) if b == 0 else (ob1, so1)
                    isb, isem = (sb1, si1) if b == 0 else (sb0, si0)

                    nq = jnp.minimum(q + 1, ppw - 1)
                    pltpu.async_copy(hbm_pair(nq), isb, isem)
                    wait_pair(buf, sem_i)
                    compute_pair(buf)

                    @pl.when(t > 0)
                    def _():
                        wait_pair(obuf, sem_o)
                    emit_pair(obuf)
                    pltpu.async_copy(obuf, out_pair(q), sem_o)
                return carry

            lax.fori_loop(0, tmax, step, 0)
            wait_pair(ob0, so0)
            wait_pair(ob1, so1)
            wait_pair(sb0, si0)

        for rr in range(max_left):
            @pl.when(ppw > 2 * tmax + rr)
            def _():
                q = 2 * tmax + rr
                pltpu.sync_copy(hbm_pair(q), sb0)
                compute_pair(sb0)
                emit_pair(ob0)
                pltpu.sync_copy(ob0, out_pair(q))

    return cluster


def kernel(x):
    shape = x.shape
    flat = x.reshape(-1)
    n = flat.shape[0]
    m = n // CACHELINE

    num_groups = m // _L
    paired_groups = (num_groups // 2) * 2
    covered = paired_groups * GROUP_ELEMS

    out = _make_cluster_call(paired_groups)(flat[:covered])

    if covered != m * CACHELINE:
        tail = flat[covered: m * CACHELINE]
        tpad = PAIR_ELEMS - tail.shape[0]
        tarr = jnp.concatenate([tail, jnp.zeros((tpad,), jnp.float32)])
        tout = _make_cluster_call(2)(tarr)
        out = jnp.concatenate([out, tout[: tail.shape[0]]])
    if m * CACHELINE != n:
        out = jnp.concatenate([out, flat[m * CACHELINE:]])
    return out.reshape(shape)
